# baseline (speedup 1.0000x reference)
import functools

import jax
import jax.numpy as jnp
from jax import lax
from jax.experimental import pallas as pl
from jax.experimental.pallas import tpu as pltpu
from jax.experimental.pallas import tpu_sc as plsc

N = 10000
E = 320000
D = 128
HD = 64
H = 64
L = 5
G = 64
C = 10
BN_EPS = 1e-5

NC = 2
NS = 16
CH = 64
EPAD = 327680
ROWS = EPAD // CH
RPW = ROWS // NS
NBUF = 4
HK = 6
ACC_N = 10240
TRASH = 10000
TPT = ACC_N // NS
DFULL = 640
DLAST = 400
IDXR = 8
GRP = RPW // (2 * IDXR)


@functools.lru_cache(maxsize=None)
def _sc_segment_sum():
    mesh = plsc.VectorSubcoreMesh(core_axis_name="c", subcore_axis_name="s")

    @functools.partial(
        pl.kernel,
        out_type=jax.ShapeDtypeStruct((NC * N, HD), jnp.float32),
        mesh=mesh,
        scratch_types=[
            pltpu.VMEM((IDXR, CH), jnp.int32),
            pltpu.VMEM((IDXR, CH), jnp.int32),
            pltpu.VMEM((IDXR, CH), jnp.int32),
            pltpu.VMEM((IDXR, CH), jnp.int32),
            [pltpu.VMEM((CH, HD), jnp.float32) for _ in range(NBUF)],
            pltpu.VMEM_SHARED((ACC_N, HD), jnp.float32),
            pltpu.VMEM_SHARED((ACC_N, HD), jnp.float32),
            [pltpu.SemaphoreType.DMA for _ in range(NBUF)],
            pltpu.SemaphoreType.DMA,
        ],
        compiler_params=pltpu.CompilerParams(use_tc_tiling_on_sc=False),
    )
    def seg_sum(h0_hbm, h1_hbm, src_hbm, dst_hbm, out_hbm, src_a, src_b,
                dst_a, dst_b, bufs, acc, hsp, sems, semi):
        c = lax.axis_index("c")
        s = lax.axis_index("s")
        base = s * RPW

        zv = jnp.zeros((16,), jnp.float32)

        def zstore(i, carry):
            bufs[0][i // (HD // 16), pl.ds((i % (HD // 16)) * 16, 16)] = zv
            return carry

        lax.fori_loop(0, CH * (HD // 16), zstore, 0)

        z0 = s * TPT

        def zcopy(i, carry):
            pltpu.sync_copy(bufs[0], acc.at[pl.ds(z0 + i * CH, CH)])
            return carry

        lax.fori_loop(0, TPT // CH, zcopy, 0)

        def edge_phase(h_hbm):
            d0 = s * DFULL

            @pl.when(s < NS - 1)
            def _():
                pltpu.sync_copy(h_hbm.at[pl.ds(d0, DFULL)],
                                hsp.at[pl.ds(d0, DFULL)])

            @pl.when(s == NS - 1)
            def _():
                pltpu.sync_copy(h_hbm.at[pl.ds(d0, DLAST)],
                                hsp.at[pl.ds(d0, DLAST)])

            plsc.subcore_barrier()

            def srow(k):
                return src_a.at[k] if k < IDXR else src_b.at[k - IDXR]

            def drow(k):
                return dst_a.at[k] if k < IDXR else dst_b.at[k - IDXR]

            def gsrc(k, row):
                if (k % IDXR) < HK:
                    return h_hbm.at[row]
                return hsp.at[row]

            pltpu.async_copy(src_hbm.at[pl.ds(base, IDXR)], src_a, semi)
            pltpu.async_copy(dst_hbm.at[pl.ds(base, IDXR)], dst_a, semi)
            pltpu.make_async_copy(
                src_hbm.at[pl.ds(base, IDXR)], src_a, semi).wait()
            pltpu.make_async_copy(
                dst_hbm.at[pl.ds(base, IDXR)], dst_a, semi).wait()
            pltpu.async_copy(src_hbm.at[pl.ds(base + IDXR, IDXR)], src_b, semi)
            pltpu.async_copy(dst_hbm.at[pl.ds(base + IDXR, IDXR)], dst_b, semi)

            for k in range(NBUF):
                pltpu.async_copy(gsrc(k, srow(k)), bufs[k], sems[k])

            def group(g, carry):
                for k in range(2 * IDXR):
                    b = k % NBUF

                    if k == NBUF:
                        pltpu.make_async_copy(
                            src_hbm.at[pl.ds(base, IDXR)], src_b, semi).wait()
                        pltpu.make_async_copy(
                            dst_hbm.at[pl.ds(base, IDXR)], dst_b, semi).wait()

                    if k == 2 * IDXR - NBUF:
                        @pl.when(g < GRP - 1)
                        def _():
                            pltpu.make_async_copy(
                                src_hbm.at[pl.ds(base, IDXR)], src_a,
                                semi).wait()
                            pltpu.make_async_copy(
                                dst_hbm.at[pl.ds(base, IDXR)], dst_a,
                                semi).wait()

                    pltpu.make_async_copy(
                        gsrc(k, srow(k)), bufs[b], sems[b]).wait()
                    pltpu.sync_copy(bufs[b], acc.at[drow(k)], add=True)

                    nk = k + NBUF
                    if nk < 2 * IDXR:
                        pltpu.async_copy(
                            gsrc(nk, srow(nk)), bufs[b], sems[b])
                    else:
                        @pl.when(g < GRP - 1)
                        def _():
                            nr = nk - 2 * IDXR
                            row = (src_a if nr < IDXR else src_b).at[nr % IDXR]
                            pltpu.async_copy(
                                gsrc(nr, row), bufs[b], sems[b])

                    if k == IDXR - 1:
                        @pl.when(g < GRP - 1)
                        def _():
                            nb2 = base + (g + 1) * 2 * IDXR
                            pltpu.async_copy(
                                src_hbm.at[pl.ds(nb2, IDXR)], src_a, semi)
                            pltpu.async_copy(
                                dst_hbm.at[pl.ds(nb2, IDXR)], dst_a, semi)

                    if k == 2 * IDXR - 1:
                        @pl.when(g < GRP - 1)
                        def _():
                            nb3 = base + (g + 1) * 2 * IDXR + IDXR
                            pltpu.async_copy(
                                src_hbm.at[pl.ds(nb3, IDXR)], src_b, semi)
                            pltpu.async_copy(
                                dst_hbm.at[pl.ds(nb3, IDXR)], dst_b, semi)
                return carry

            lax.fori_loop(0, GRP, group, 0)

        @pl.when(c == 0)
        def _():
            edge_phase(h0_hbm)

        @pl.when(c == 1)
        def _():
            edge_phase(h1_hbm)

        plsc.subcore_barrier()

        d0 = s * DFULL

        @pl.when(s < NS - 1)
        def _():
            pltpu.sync_copy(acc.at[pl.ds(d0, DFULL)],
                            out_hbm.at[pl.ds(c * N + d0, DFULL)])

        @pl.when(s == NS - 1)
        def _():
            pltpu.sync_copy(acc.at[pl.ds(d0, DLAST)],
                            out_hbm.at[pl.ds(c * N + d0, DLAST)])

    return seg_sum


def _mlp_body(relu_out, split_out, h0_ref, h1_ref, a0_ref, a1_ref, w1_ref,
              b1_ref, w2_ref, b2_ref, *o_refs):
    z = jnp.concatenate(
        [h0_ref[...] + a0_ref[...], h1_ref[...] + a1_ref[...]], axis=1)
    z = jnp.dot(z, w1_ref[...], preferred_element_type=jnp.float32)
    z = jnp.maximum(z + b1_ref[...], 0.0)
    z = jnp.dot(z, w2_ref[...], preferred_element_type=jnp.float32)
    z = z + b2_ref[...]
    if relu_out:
        z = jnp.maximum(z, 0.0)
    if split_out:
        o_refs[0][...] = z[:, :HD]
        o_refs[1][...] = z[:, HD:]
    else:
        o_refs[0][...] = z


@functools.lru_cache(maxsize=None)
def _tc_layer(relu_out, split_out):
    tb = 1000
    nb = N // tb
    if split_out:
        out_shape = [jax.ShapeDtypeStruct((N, HD), jnp.float32),
                     jax.ShapeDtypeStruct((N, HD), jnp.float32)]
        out_specs = [pl.BlockSpec((tb, HD), lambda i: (i, 0)),
                     pl.BlockSpec((tb, HD), lambda i: (i, 0))]
    else:
        out_shape = jax.ShapeDtypeStruct((N, D), jnp.float32)
        out_specs = pl.BlockSpec((tb, D), lambda i: (i, 0))
    return pl.pallas_call(
        functools.partial(_mlp_body, relu_out, split_out),
        grid=(nb,),
        in_specs=[
            pl.BlockSpec((tb, HD), lambda i: (i, 0)),
            pl.BlockSpec((tb, HD), lambda i: (i, 0)),
            pl.BlockSpec((tb, HD), lambda i: (i, 0)),
            pl.BlockSpec((tb, HD), lambda i: (i + nb, 0)),
            pl.BlockSpec((D, H), lambda i: (0, 0)),
            pl.BlockSpec((1, H), lambda i: (0, 0)),
            pl.BlockSpec((H, D), lambda i: (0, 0)),
            pl.BlockSpec((1, D), lambda i: (0, 0)),
        ],
        out_specs=out_specs,
        out_shape=out_shape,
    )


def _pool_body(h_ref, b_ref, wm_ref, bm_ref, o_ref, acc_ref, cnt_ref):
    i = pl.program_id(0)

    @pl.when(i == 0)
    def _():
        acc_ref[...] = jnp.zeros_like(acc_ref)
        cnt_ref[...] = jnp.zeros_like(cnt_ref)

    p = (b_ref[...] == lax.broadcasted_iota(jnp.int32, (1, G), 1))
    p = p.astype(jnp.float32)
    acc_ref[...] += lax.dot_general(
        p, h_ref[...], (((0,), (0,)), ((), ())),
        preferred_element_type=jnp.float32)
    cnt_ref[...] += lax.dot_general(
        p, jnp.ones((p.shape[0], 1), jnp.float32), (((0,), (0,)), ((), ())),
        preferred_element_type=jnp.float32)

    @pl.when(i == pl.num_programs(0) - 1)
    def _():
        hg = acc_ref[...] / jnp.maximum(cnt_ref[...], 1.0)
        o_ref[...] = jnp.dot(hg, wm_ref[...],
                             preferred_element_type=jnp.float32) + bm_ref[...]


@functools.lru_cache(maxsize=None)
def _tc_pool():
    tb = 1000
    nb = N // tb
    return pl.pallas_call(
        _pool_body,
        grid=(nb,),
        in_specs=[
            pl.BlockSpec((tb, D), lambda i: (i, 0)),
            pl.BlockSpec((tb, 1), lambda i: (i, 0)),
            pl.BlockSpec((D, C), lambda i: (0, 0)),
            pl.BlockSpec((1, C), lambda i: (0, 0)),
        ],
        out_specs=pl.BlockSpec((G, C), lambda i: (0, 0)),
        out_shape=jax.ShapeDtypeStruct((G, C), jnp.float32),
        scratch_shapes=[
            pltpu.VMEM((G, D), jnp.float32),
            pltpu.VMEM((G, 1), jnp.float32),
        ],
    )


def kernel(x, edge_index, batch, W1, b1, g1, be1, W2, b2, g2, be2, Wm, bm):
    s = 1.0 / jnp.sqrt(1.0 + BN_EPS)
    w1f = W1 * (g1 * s)[:, None, :]
    b1f = b1 * g1 * s + be1
    w2f = W2 * (g2 * s)[:, None, :]
    b2f = b2 * g2 * s + be2

    pad = EPAD - E
    padsrc = (jnp.arange(pad, dtype=jnp.int32) * 131) % N
    src2d = jnp.concatenate([edge_index[0], padsrc]).reshape(ROWS, CH)
    trash = TRASH + jnp.arange(pad, dtype=jnp.int32) % (ACC_N - TRASH)
    dst2d = jnp.concatenate([edge_index[1], trash]).reshape(ROWS, CH)

    seg_sum = _sc_segment_sum()
    h0 = x[:, :HD]
    h1 = x[:, HD:]
    for l in range(L):
        agg = seg_sum(h0, h1, src2d, dst2d)
        if l < L - 1:
            h0, h1 = _tc_layer(True, True)(
                h0, h1, agg, agg, w1f[l], b1f[l][None], w2f[l], b2f[l][None])
        else:
            h = _tc_layer(False, False)(
                h0, h1, agg, agg, w1f[l], b1f[l][None], w2f[l], b2f[l][None])

    return _tc_pool()(h, batch[:, None], Wm, bm[None])


# speedup vs baseline: 1.5521x; 1.5521x over previous
import functools

import jax
import jax.numpy as jnp
from jax import lax
from jax.experimental import pallas as pl
from jax.experimental.pallas import tpu as pltpu
from jax.experimental.pallas import tpu_sc as plsc

N = 10000
E = 320000
D = 128
H = 64
L = 5
G = 64
C = 10
BN_EPS = 1e-5

NC = 2
NS = 16
NW = NC * NS
CH = 64
EPAD = 327680
ROWS = EPAD // CH
RPW = ROWS // NW
NBUF = 4
ACC_N = 10240
TRASH = 10000
TPT = ACC_N // NS
DFULL = 640
DLAST = 400
IDXR = 8
GRP = RPW // (2 * IDXR)


@functools.lru_cache(maxsize=None)
def _sc_segment_sum():
    mesh = plsc.VectorSubcoreMesh(core_axis_name="c", subcore_axis_name="s")

    @functools.partial(
        pl.kernel,
        out_type=jax.ShapeDtypeStruct((NC * N, D), jnp.float32),
        mesh=mesh,
        scratch_types=[
            pltpu.VMEM((IDXR, CH), jnp.int32),
            pltpu.VMEM((IDXR, CH), jnp.int32),
            pltpu.VMEM((IDXR, CH), jnp.int32),
            pltpu.VMEM((IDXR, CH), jnp.int32),
            [pltpu.VMEM((CH, D), jnp.float32) for _ in range(NBUF)],
            pltpu.VMEM_SHARED((ACC_N, D), jnp.float32),
            [pltpu.SemaphoreType.DMA for _ in range(NBUF)],
            pltpu.SemaphoreType.DMA,
        ],
    )
    def seg_sum(h_hbm, src_hbm, dst_hbm, out_hbm, src_a, src_b, dst_a, dst_b,
                bufs, acc, sems, semi):
        c = lax.axis_index("c")
        s = lax.axis_index("s")
        wid = c * NS + s
        base = wid * RPW

        zv = jnp.zeros((16,), jnp.float32)

        def zstore(i, carry):
            bufs[0][i // 8, pl.ds((i % 8) * 16, 16)] = zv
            return carry

        lax.fori_loop(0, CH * 8, zstore, 0)

        z0 = s * TPT

        def zcopy(i, carry):
            pltpu.sync_copy(bufs[0], acc.at[pl.ds(z0 + i * CH, CH)])
            return carry

        lax.fori_loop(0, TPT // CH, zcopy, 0)
        plsc.subcore_barrier()

        def srow(k):
            return src_a.at[k] if k < IDXR else src_b.at[k - IDXR]

        def drow(k):
            return dst_a.at[k] if k < IDXR else dst_b.at[k - IDXR]

        pltpu.async_copy(src_hbm.at[pl.ds(base, IDXR)], src_a, semi)
        pltpu.async_copy(dst_hbm.at[pl.ds(base, IDXR)], dst_a, semi)
        pltpu.make_async_copy(
            src_hbm.at[pl.ds(base, IDXR)], src_a, semi).wait()
        pltpu.make_async_copy(
            dst_hbm.at[pl.ds(base, IDXR)], dst_a, semi).wait()
        pltpu.async_copy(src_hbm.at[pl.ds(base + IDXR, IDXR)], src_b, semi)
        pltpu.async_copy(dst_hbm.at[pl.ds(base + IDXR, IDXR)], dst_b, semi)

        for k in range(NBUF):
            pltpu.async_copy(h_hbm.at[srow(k)], bufs[k], sems[k])

        def group(g, carry):
            for k in range(2 * IDXR):
                b = k % NBUF

                if k == NBUF:
                    pltpu.make_async_copy(
                        src_hbm.at[pl.ds(base, IDXR)], src_b, semi).wait()
                    pltpu.make_async_copy(
                        dst_hbm.at[pl.ds(base, IDXR)], dst_b, semi).wait()

                if k == 2 * IDXR - NBUF:
                    @pl.when(g < GRP - 1)
                    def _():
                        pltpu.make_async_copy(
                            src_hbm.at[pl.ds(base, IDXR)], src_a, semi).wait()
                        pltpu.make_async_copy(
                            dst_hbm.at[pl.ds(base, IDXR)], dst_a, semi).wait()

                pltpu.make_async_copy(
                    h_hbm.at[srow(k)], bufs[b], sems[b]).wait()
                pltpu.sync_copy(bufs[b], acc.at[drow(k)], add=True)

                nk = k + NBUF
                if nk < 2 * IDXR:
                    pltpu.async_copy(h_hbm.at[srow(nk)], bufs[b], sems[b])
                else:
                    @pl.when(g < GRP - 1)
                    def _():
                        nr = nk - 2 * IDXR
                        off = base + (g + 1) * 2 * IDXR + (nr // IDXR) * IDXR
                        row = (src_a if nr < IDXR else src_b).at[nr % IDXR]
                        pltpu.async_copy(h_hbm.at[row], bufs[b], sems[b])

                if k == IDXR - 1:
                    @pl.when(g < GRP - 1)
                    def _():
                        nb2 = base + (g + 1) * 2 * IDXR
                        pltpu.async_copy(
                            src_hbm.at[pl.ds(nb2, IDXR)], src_a, semi)
                        pltpu.async_copy(
                            dst_hbm.at[pl.ds(nb2, IDXR)], dst_a, semi)

                if k == 2 * IDXR - 1:
                    @pl.when(g < GRP - 1)
                    def _():
                        nb3 = base + (g + 1) * 2 * IDXR + IDXR
                        pltpu.async_copy(
                            src_hbm.at[pl.ds(nb3, IDXR)], src_b, semi)
                        pltpu.async_copy(
                            dst_hbm.at[pl.ds(nb3, IDXR)], dst_b, semi)
            return carry

        lax.fori_loop(0, GRP, group, 0)
        plsc.subcore_barrier()

        d0 = s * DFULL

        @pl.when(s < NS - 1)
        def _():
            pltpu.sync_copy(acc.at[pl.ds(d0, DFULL)],
                            out_hbm.at[pl.ds(c * N + d0, DFULL)])

        @pl.when(s == NS - 1)
        def _():
            pltpu.sync_copy(acc.at[pl.ds(d0, DLAST)],
                            out_hbm.at[pl.ds(c * N + d0, DLAST)])

    return seg_sum


def _mlp_body(relu_out, h_ref, a0_ref, a1_ref, w1_ref, b1_ref, w2_ref, b2_ref,
              o_ref):
    z = h_ref[...] + a0_ref[...] + a1_ref[...]
    z = jnp.dot(z, w1_ref[...], preferred_element_type=jnp.float32)
    z = jnp.maximum(z + b1_ref[...], 0.0)
    z = jnp.dot(z, w2_ref[...], preferred_element_type=jnp.float32)
    z = z + b2_ref[...]
    if relu_out:
        z = jnp.maximum(z, 0.0)
    o_ref[...] = z


@functools.lru_cache(maxsize=None)
def _tc_layer(relu_out):
    tb = 2000
    nb = N // tb
    return pl.pallas_call(
        functools.partial(_mlp_body, relu_out),
        grid=(nb,),
        in_specs=[
            pl.BlockSpec((tb, D), lambda i: (i, 0)),
            pl.BlockSpec((tb, D), lambda i: (i, 0)),
            pl.BlockSpec((tb, D), lambda i: (i + nb, 0)),
            pl.BlockSpec((D, H), lambda i: (0, 0)),
            pl.BlockSpec((1, H), lambda i: (0, 0)),
            pl.BlockSpec((H, D), lambda i: (0, 0)),
            pl.BlockSpec((1, D), lambda i: (0, 0)),
        ],
        out_specs=pl.BlockSpec((tb, D), lambda i: (i, 0)),
        out_shape=jax.ShapeDtypeStruct((N, D), jnp.float32),
    )


def _mlp_pool_body(h_ref, a0_ref, a1_ref, w1_ref, b1_ref, w2_ref, b2_ref,
                   b_ref, wm_ref, bm_ref, o_ref, acc_ref, cnt_ref):
    i = pl.program_id(0)

    @pl.when(i == 0)
    def _():
        acc_ref[...] = jnp.zeros_like(acc_ref)
        cnt_ref[...] = jnp.zeros_like(cnt_ref)

    z = h_ref[...] + a0_ref[...] + a1_ref[...]
    z = jnp.dot(z, w1_ref[...], preferred_element_type=jnp.float32)
    z = jnp.maximum(z + b1_ref[...], 0.0)
    z = jnp.dot(z, w2_ref[...], preferred_element_type=jnp.float32)
    z = z + b2_ref[...]

    p = (b_ref[...] == lax.broadcasted_iota(jnp.int32, (1, G), 1))
    p = p.astype(jnp.float32)
    acc_ref[...] += lax.dot_general(
        p, z, (((0,), (0,)), ((), ())),
        preferred_element_type=jnp.float32)
    cnt_ref[...] += lax.dot_general(
        p, jnp.ones((p.shape[0], 1), jnp.float32), (((0,), (0,)), ((), ())),
        preferred_element_type=jnp.float32)

    @pl.when(i == pl.num_programs(0) - 1)
    def _():
        hg = acc_ref[...] / jnp.maximum(cnt_ref[...], 1.0)
        o_ref[...] = jnp.dot(hg, wm_ref[...],
                             preferred_element_type=jnp.float32) + bm_ref[...]


@functools.lru_cache(maxsize=None)
def _tc_last_layer_pool():
    tb = 2000
    nb = N // tb
    return pl.pallas_call(
        _mlp_pool_body,
        grid=(nb,),
        in_specs=[
            pl.BlockSpec((tb, D), lambda i: (i, 0)),
            pl.BlockSpec((tb, D), lambda i: (i, 0)),
            pl.BlockSpec((tb, D), lambda i: (i + nb, 0)),
            pl.BlockSpec((D, H), lambda i: (0, 0)),
            pl.BlockSpec((1, H), lambda i: (0, 0)),
            pl.BlockSpec((H, D), lambda i: (0, 0)),
            pl.BlockSpec((1, D), lambda i: (0, 0)),
            pl.BlockSpec((tb, 1), lambda i: (i, 0)),
            pl.BlockSpec((D, C), lambda i: (0, 0)),
            pl.BlockSpec((1, C), lambda i: (0, 0)),
        ],
        out_specs=pl.BlockSpec((G, C), lambda i: (0, 0)),
        out_shape=jax.ShapeDtypeStruct((G, C), jnp.float32),
        scratch_shapes=[
            pltpu.VMEM((G, D), jnp.float32),
            pltpu.VMEM((G, 1), jnp.float32),
        ],
    )


def kernel(x, edge_index, batch, W1, b1, g1, be1, W2, b2, g2, be2, Wm, bm):
    s = 1.0 / jnp.sqrt(1.0 + BN_EPS)
    w1f = W1 * (g1 * s)[:, None, :]
    b1f = b1 * g1 * s + be1
    w2f = W2 * (g2 * s)[:, None, :]
    b2f = b2 * g2 * s + be2

    pad = EPAD - E
    padsrc = (jnp.arange(pad, dtype=jnp.int32) * 131) % N
    src2d = jnp.concatenate([edge_index[0], padsrc]).reshape(ROWS, CH)
    trash = TRASH + jnp.arange(pad, dtype=jnp.int32) % (ACC_N - TRASH)
    dst2d = jnp.concatenate([edge_index[1], trash]).reshape(ROWS, CH)

    seg_sum = _sc_segment_sum()
    h = x
    for l in range(L - 1):
        agg = seg_sum(h, src2d, dst2d)
        h = _tc_layer(True)(
            h, agg, agg, w1f[l], b1f[l][None], w2f[l], b2f[l][None])

    agg = seg_sum(h, src2d, dst2d)
    return _tc_last_layer_pool()(
        h, agg, agg, w1f[L - 1], b1f[L - 1][None], w2f[L - 1], b2f[L - 1][None],
        batch[:, None], Wm, bm[None])
